# baseline (speedup 1.0000x reference)
import functools

import jax
import jax.numpy as jnp
from jax import lax
from jax.experimental import pallas as pl
from jax.experimental.pallas import tpu as pltpu
from jax.experimental.pallas import tpu_sc as plsc

_SIGMA = 9.0
_N = 2097152
_NC = 2
_NS = 16
_NW = _NC * _NS
_RW = _N // _NW
_CH = 8192
_NCH = _RW // _CH
_U = 4


def _sc_body(pred_hbm, lbl_hbm, out_hbm,
             pbuf0, pbuf1, lbuf0, lbuf1, ssum, scnt, sem0, sem1):
    cid = lax.axis_index("c")
    sid = lax.axis_index("s")
    wid = cid * _NS + sid

    pbufs = (pbuf0, pbuf1)
    lbufs = (lbuf0, lbuf1)
    sems = (sem0, sem1)

    p_base = wid * (_RW * 2)
    l_base = wid * (_RW * 3)

    def start(c, b):
        h1 = pltpu.async_copy(
            pred_hbm.at[pl.ds(p_base + c * (_CH * 2), _CH * 2)],
            pbufs[b], sems[b])
        h2 = pltpu.async_copy(
            lbl_hbm.at[pl.ds(l_base + c * (_CH * 3), _CH * 3)],
            lbufs[b], sems[b])
        return (h1, h2)

    iota = lax.iota(jnp.int32, 16)
    i3 = iota * 3
    i2 = iota * 2

    def group(pbuf, lbuf, bl, bp, s, cn):
        cls = plsc.load_gather(lbuf, [bl + i3])
        o0 = plsc.load_gather(lbuf, [bl + (i3 + 1)])
        o1 = plsc.load_gather(lbuf, [bl + (i3 + 2)])
        p0 = plsc.load_gather(pbuf, [bp + i2])
        p1 = plsc.load_gather(pbuf, [bp + (i2 + 1)])
        d0 = jnp.abs(o0 - p0)
        d1 = jnp.abs(o1 - p1)
        h0 = jnp.where(d0 < (1.0 / _SIGMA),
                       (0.5 * _SIGMA) * d0 * d0, d0 - 0.5 / _SIGMA)
        h1 = jnp.where(d1 < (1.0 / _SIGMA),
                       (0.5 * _SIGMA) * d1 * d1, d1 - 0.5 / _SIGMA)
        msk = cls == 1.0
        s = s + jnp.where(msk, h0 + h1, 0.0)
        cn = cn + jnp.where(msk, 1.0, 0.0)
        return s, cn

    def compute(b, s, cn):
        pbuf = pbufs[b]
        lbuf = lbufs[b]

        def body(m, carry):
            s, cn = carry
            bl = m * (48 * _U)
            bp = m * (32 * _U)
            for u in range(_U):
                s, cn = group(pbuf, lbuf, bl + 48 * u, bp + 32 * u, s, cn)
            return s, cn

        return lax.fori_loop(0, _CH // (16 * _U), body, (s, cn))

    s = jnp.zeros((16,), jnp.float32)
    cn = jnp.zeros((16,), jnp.float32)
    handles = start(0, 0)
    for c in range(_NCH):
        if c + 1 < _NCH:
            nxt = start(c + 1, (c + 1) % 2)
        for h in handles:
            h.wait()
        s, cn = compute(c % 2, s, cn)
        if c + 1 < _NCH:
            handles = nxt

    ssum[...] = s
    scnt[...] = cn
    pltpu.sync_copy(ssum, out_hbm.at[wid])
    pltpu.sync_copy(scnt, out_hbm.at[_NW + wid])


@jax.jit
def _run(pred_flat, lbl_flat):
    mesh = plsc.VectorSubcoreMesh(core_axis_name="c", subcore_axis_name="s")
    k = functools.partial(
        pl.kernel,
        out_type=jax.ShapeDtypeStruct((2 * _NW, 16), jnp.float32),
        mesh=mesh,
        compiler_params=pltpu.CompilerParams(needs_layout_passes=False),
        scratch_types=[
            pltpu.VMEM((_CH * 2,), jnp.float32),
            pltpu.VMEM((_CH * 2,), jnp.float32),
            pltpu.VMEM((_CH * 3,), jnp.float32),
            pltpu.VMEM((_CH * 3,), jnp.float32),
            pltpu.VMEM((16,), jnp.float32),
            pltpu.VMEM((16,), jnp.float32),
            pltpu.SemaphoreType.DMA,
            pltpu.SemaphoreType.DMA,
        ],
    )(_sc_body)
    partials = k(pred_flat, lbl_flat)
    total = jnp.sum(partials[:_NW])
    cnt = jnp.sum(partials[_NW:])
    return jnp.where(cnt > 0, total / jnp.maximum(cnt, 1.0),
                     jnp.float32(0.0))


def kernel(输入, 标签):
    pred_flat = 输入.reshape(_N * 2)
    lbl_flat = 标签.reshape(_N * 3)
    return _run(pred_flat, lbl_flat)
